# Initial kernel scaffold; baseline (speedup 1.0000x reference)
import jax
import jax.numpy as jnp
from jax.experimental import pallas as pl


def kernel(emb, w_relation, src_pos, dst_pos, edge_type, src_neg, dst_neg):
    raise NotImplementedError("write your pallas kernel here")


# speedup vs baseline: 1.6423x; 1.6423x over previous
import functools

import jax
import jax.numpy as jnp
from jax import lax
from jax.experimental import pallas as pl
from jax.experimental.pallas import tpu as pltpu
from jax.experimental.pallas import tpu_sc as plsc

N_NODES = 1000000
H = 64
NUM_RELS = 1000
B = 16384
NEG = 20
BN = B * NEG

NC = 2
NS = 16
L = 16
NW = NC * NS

POS_PER_W = B // NW
NEG_PER_W = BN // NW
CHUNK = 640
NCHUNK = NEG_PER_W // CHUNK
KSUB = H // L


def _score_rows(heads, tails, w_for_row, scores, psum, t):
    lane = jnp.arange(L, dtype=jnp.int32)
    rbase = t * 16
    for p in range(16):
        rl = rbase + p
        acc = jnp.zeros((L,), jnp.float32)
        for k in range(KSUB):
            hk = heads[rl, pl.ds(16 * k, 16)]
            tk = tails[rl, pl.ds(16 * k, 16)]
            wk = w_for_row(rl, k, lane)
            acc = acc + hk * wk * tk
        psum[p, :] = acc
    svec = jnp.zeros((L,), jnp.float32)
    for j in range(L):
        svec = svec + plsc.load_gather(
            psum, [lane, jnp.full((L,), j, jnp.int32)])
    scores[pl.ds(rbase, 16)] = svec
    return 0


def _sc_body(emb, w_rel, src_pos, dst_pos, etype, src_neg, dst_neg,
             pos_out, neg_out,
             hidx, tidx, wrows, heads, tails, scores, psum, sem):
    wid = lax.axis_index("s") * NC + lax.axis_index("c")

    pos0 = wid * POS_PER_W
    pltpu.sync_copy(etype.at[pl.ds(pos0, POS_PER_W)],
                    hidx.at[pl.ds(0, POS_PER_W)])
    cps = []
    for j in range(POS_PER_W // 128):
        cps.append(pltpu.async_copy(
            w_rel.at[hidx.at[pl.ds(j * 128, 128)]],
            wrows.at[pl.ds(j * 128, 128)], sem))
    for c in cps:
        c.wait()

    pltpu.sync_copy(src_pos.at[pl.ds(pos0, POS_PER_W)],
                    hidx.at[pl.ds(0, POS_PER_W)])
    pltpu.sync_copy(dst_pos.at[pl.ds(pos0, POS_PER_W)],
                    tidx.at[pl.ds(0, POS_PER_W)])
    cps = []
    for j in range(POS_PER_W // 128):
        cps.append(pltpu.async_copy(
            emb.at[hidx.at[pl.ds(j * 128, 128)]],
            heads.at[pl.ds(j * 128, 128)], sem))
        cps.append(pltpu.async_copy(
            emb.at[tidx.at[pl.ds(j * 128, 128)]],
            tails.at[pl.ds(j * 128, 128)], sem))
    for c in cps:
        c.wait()

    def w_pos(rl, k, lane):
        return wrows[rl, pl.ds(16 * k, 16)]

    def pos_tile(t, carry):
        return _score_rows(heads, tails, w_pos, scores, psum, t)

    lax.fori_loop(0, POS_PER_W // 16, pos_tile, 0)
    pltpu.sync_copy(scores.at[pl.ds(0, POS_PER_W)],
                    pos_out.at[pl.ds(wid * POS_PER_W, POS_PER_W)])

    def neg_chunk(c, carry):
        neg0 = wid * NEG_PER_W + c * CHUNK
        pltpu.sync_copy(src_neg.at[pl.ds(neg0, CHUNK)], hidx)
        pltpu.sync_copy(dst_neg.at[pl.ds(neg0, CHUNK)], tidx)
        cps2 = []
        for j in range(CHUNK // 128):
            cps2.append(pltpu.async_copy(
                emb.at[hidx.at[pl.ds(j * 128, 128)]],
                heads.at[pl.ds(j * 128, 128)], sem))
            cps2.append(pltpu.async_copy(
                emb.at[tidx.at[pl.ds(j * 128, 128)]],
                tails.at[pl.ds(j * 128, 128)], sem))
        for cp in cps2:
            cp.wait()

        def w_neg(rl, k, lane):
            fvec = lax.div(64 * (c * CHUNK + rl) + 16 * k + lane,
                           jnp.int32(20))
            return plsc.load_gather(wrows, [fvec >> 6, fvec & 63])

        def neg_tile(t, carry2):
            return _score_rows(heads, tails, w_neg, scores, psum, t)

        lax.fori_loop(0, CHUNK // 16, neg_tile, 0)
        pltpu.sync_copy(
            scores.at[pl.ds(0, CHUNK)],
            neg_out.at[pl.ds(wid * NEG_PER_W + c * CHUNK, CHUNK)])
        return carry

    lax.fori_loop(0, NCHUNK, neg_chunk, 0)


_sc_score = functools.partial(
    pl.kernel,
    out_type=(jax.ShapeDtypeStruct((B,), jnp.float32),
              jax.ShapeDtypeStruct((BN,), jnp.float32)),
    mesh=plsc.VectorSubcoreMesh(core_axis_name="c", subcore_axis_name="s"),
    compiler_params=pltpu.CompilerParams(needs_layout_passes=False,
                                         use_tc_tiling_on_sc=False),
    scratch_types=(
        pltpu.VMEM((CHUNK,), jnp.int32),
        pltpu.VMEM((CHUNK,), jnp.int32),
        pltpu.VMEM((POS_PER_W, H), jnp.float32),
        pltpu.VMEM((CHUNK, H), jnp.float32),
        pltpu.VMEM((CHUNK, H), jnp.float32),
        pltpu.VMEM((CHUNK,), jnp.float32),
        pltpu.VMEM((L, L), jnp.float32),
        pltpu.SemaphoreType.DMA,
    ),
)(_sc_body)


def _loss_body(pos_ref, neg_ref, out_ref):
    def softplus(z):
        return jnp.maximum(z, 0.0) + jnp.log1p(jnp.exp(-jnp.abs(z)))

    total = jnp.sum(softplus(-pos_ref[...])) + jnp.sum(softplus(neg_ref[...]))
    out_ref[0, 0] = total / B


def kernel(emb, w_relation, src_pos, dst_pos, edge_type, src_neg, dst_neg):
    i32 = jnp.int32
    pos_score, neg_score = _sc_score(
        emb, w_relation, src_pos.astype(i32), dst_pos.astype(i32),
        edge_type.astype(i32), src_neg.astype(i32), dst_neg.astype(i32))

    loss = pl.pallas_call(
        _loss_body,
        out_shape=jax.ShapeDtypeStruct((1, 1), jnp.float32),
        out_specs=pl.BlockSpec(memory_space=pltpu.SMEM),
    )(pos_score.reshape(B // 128, 128), neg_score.reshape(BN // 128, 128))
    return loss[0, 0]


# speedup vs baseline: 1.7985x; 1.0951x over previous
import functools

import jax
import jax.numpy as jnp
from jax import lax
from jax.experimental import pallas as pl
from jax.experimental.pallas import tpu as pltpu
from jax.experimental.pallas import tpu_sc as plsc

N_NODES = 1000000
H = 64
NUM_RELS = 1000
B = 16384
NEG = 20
BN = B * NEG

NC = 2
NS = 16
L = 16
NW = NC * NS

POS_PER_W = B // NW
NEG_PER_W = BN // NW
CHUNK = 256
NCHUNK = NEG_PER_W // CHUNK
NPAIR = NCHUNK // 2
KSUB = H // L


def _score_rows(heads, tails, w_for_row, scores, psum, t, score_off):
    lane = jnp.arange(L, dtype=jnp.int32)
    rbase = t * 16
    for p in range(16):
        rl = rbase + p
        acc = jnp.zeros((L,), jnp.float32)
        for k in range(KSUB):
            hk = heads[rl, pl.ds(16 * k, 16)]
            tk = tails[rl, pl.ds(16 * k, 16)]
            wk = w_for_row(rl, k, lane)
            acc = acc + hk * wk * tk
        psum[p, :] = acc
    svec = jnp.zeros((L,), jnp.float32)
    for j in range(L):
        svec = svec + plsc.load_gather(
            psum, [lane, jnp.full((L,), j, jnp.int32)])
    scores[pl.ds(score_off + rbase, 16)] = svec
    return 0


def _sc_body(emb, w_rel, src_pos, dst_pos, etype, src_neg, dst_neg,
             pos_out, neg_out,
             pidx, didx, eidx, hidx0, hidx1, tidx0, tidx1,
             wrows, heads0, heads1, tails0, tails1,
             pos_sc, neg_sc, psum,
             sem_w, sem_g0, sem_g1, sem_i0, sem_i1):
    wid = lax.axis_index("s") * NC + lax.axis_index("c")
    pos0 = wid * POS_PER_W
    neg_base = wid * NEG_PER_W

    pltpu.sync_copy(etype.at[pl.ds(pos0, POS_PER_W)], eidx)
    for j in range(POS_PER_W // 128):
        pltpu.async_copy(w_rel.at[eidx.at[pl.ds(j * 128, 128)]],
                         wrows.at[pl.ds(j * 128, 128)], sem_w)
    pltpu.sync_copy(src_pos.at[pl.ds(pos0, POS_PER_W)], pidx)
    pltpu.sync_copy(dst_pos.at[pl.ds(pos0, POS_PER_W)], didx)
    for j in range(POS_PER_W // 128):
        pltpu.make_async_copy(w_rel.at[eidx.at[pl.ds(j * 128, 128)]],
                              wrows.at[pl.ds(j * 128, 128)], sem_w).wait()

    for sub in range(2):
        for j in range(2):
            pltpu.async_copy(
                emb.at[pidx.at[pl.ds(sub * 256 + j * 128, 128)]],
                heads0.at[pl.ds(j * 128, 128)], sem_g0)
            pltpu.async_copy(
                emb.at[didx.at[pl.ds(sub * 256 + j * 128, 128)]],
                tails0.at[pl.ds(j * 128, 128)], sem_g0)
        for j in range(2):
            pltpu.make_async_copy(
                emb.at[pidx.at[pl.ds(sub * 256 + j * 128, 128)]],
                heads0.at[pl.ds(j * 128, 128)], sem_g0).wait()
            pltpu.make_async_copy(
                emb.at[didx.at[pl.ds(sub * 256 + j * 128, 128)]],
                tails0.at[pl.ds(j * 128, 128)], sem_g0).wait()

        def w_pos(rl, k, lane, _sub=sub):
            return wrows[_sub * 256 + rl, pl.ds(16 * k, 16)]

        def pos_tile(t, carry, _sub=sub):
            return _score_rows(heads0, tails0, w_pos, pos_sc, psum, t,
                               _sub * 256)

        lax.fori_loop(0, 256 // 16, pos_tile, 0)
    pltpu.sync_copy(pos_sc, pos_out.at[pl.ds(pos0, POS_PER_W)])

    def fire_idx(c, hb, tb, sem):
        off = neg_base + c * CHUNK
        pltpu.async_copy(src_neg.at[pl.ds(off, CHUNK)], hb, sem)
        pltpu.async_copy(dst_neg.at[pl.ds(off, CHUNK)], tb, sem)

    def wait_idx(hb, tb, sem):
        pltpu.make_async_copy(src_neg.at[pl.ds(0, CHUNK)], hb, sem).wait()
        pltpu.make_async_copy(dst_neg.at[pl.ds(0, CHUNK)], tb, sem).wait()

    def fire_g(hb, tb, hd, td, sem):
        for j in range(2):
            pltpu.async_copy(emb.at[hb.at[pl.ds(j * 128, 128)]],
                             hd.at[pl.ds(j * 128, 128)], sem)
            pltpu.async_copy(emb.at[tb.at[pl.ds(j * 128, 128)]],
                             td.at[pl.ds(j * 128, 128)], sem)

    def wait_g(hb, tb, hd, td, sem):
        for j in range(2):
            pltpu.make_async_copy(emb.at[hb.at[pl.ds(j * 128, 128)]],
                                  hd.at[pl.ds(j * 128, 128)], sem).wait()
            pltpu.make_async_copy(emb.at[tb.at[pl.ds(j * 128, 128)]],
                                  td.at[pl.ds(j * 128, 128)], sem).wait()

    def compute(c, hd, td):
        def w_neg(rl, k, lane):
            fvec = lax.div(64 * (c * CHUNK + rl) + 16 * k + lane,
                           jnp.int32(20))
            return plsc.load_gather(wrows, [fvec >> 6, fvec & 63])

        def neg_tile(t, carry):
            return _score_rows(hd, td, w_neg, neg_sc, psum, t, c * CHUNK)

        lax.fori_loop(0, CHUNK // 16, neg_tile, 0)

    fire_idx(0, hidx0, tidx0, sem_i0)
    fire_idx(1, hidx1, tidx1, sem_i1)
    wait_idx(hidx0, tidx0, sem_i0)
    fire_g(hidx0, tidx0, heads0, tails0, sem_g0)
    wait_idx(hidx1, tidx1, sem_i1)

    def pair(i, carry):
        c0 = 2 * i
        c1 = c0 + 1
        not_last = i < NPAIR - 1
        fire_g(hidx1, tidx1, heads1, tails1, sem_g1)
        wait_g(hidx0, tidx0, heads0, tails0, sem_g0)

        @pl.when(not_last)
        def _():
            fire_idx(c0 + 2, hidx0, tidx0, sem_i0)

        compute(c0, heads0, tails0)

        @pl.when(not_last)
        def _():
            wait_idx(hidx0, tidx0, sem_i0)
            fire_g(hidx0, tidx0, heads0, tails0, sem_g0)

        wait_g(hidx1, tidx1, heads1, tails1, sem_g1)

        @pl.when(not_last)
        def _():
            fire_idx(c1 + 2, hidx1, tidx1, sem_i1)

        compute(c1, heads1, tails1)

        @pl.when(not_last)
        def _():
            wait_idx(hidx1, tidx1, sem_i1)

        return carry

    lax.fori_loop(0, NPAIR, pair, 0)
    pltpu.sync_copy(neg_sc, neg_out.at[pl.ds(neg_base, NEG_PER_W)])


_sc_score = functools.partial(
    pl.kernel,
    out_type=(jax.ShapeDtypeStruct((B,), jnp.float32),
              jax.ShapeDtypeStruct((BN,), jnp.float32)),
    mesh=plsc.VectorSubcoreMesh(core_axis_name="c", subcore_axis_name="s"),
    compiler_params=pltpu.CompilerParams(needs_layout_passes=False,
                                         use_tc_tiling_on_sc=False),
    scratch_types=(
        pltpu.VMEM((POS_PER_W,), jnp.int32),
        pltpu.VMEM((POS_PER_W,), jnp.int32),
        pltpu.VMEM((POS_PER_W,), jnp.int32),
        pltpu.VMEM((CHUNK,), jnp.int32),
        pltpu.VMEM((CHUNK,), jnp.int32),
        pltpu.VMEM((CHUNK,), jnp.int32),
        pltpu.VMEM((CHUNK,), jnp.int32),
        pltpu.VMEM((POS_PER_W, H), jnp.float32),
        pltpu.VMEM((CHUNK, H), jnp.float32),
        pltpu.VMEM((CHUNK, H), jnp.float32),
        pltpu.VMEM((CHUNK, H), jnp.float32),
        pltpu.VMEM((CHUNK, H), jnp.float32),
        pltpu.VMEM((POS_PER_W,), jnp.float32),
        pltpu.VMEM((NEG_PER_W,), jnp.float32),
        pltpu.VMEM((L, L), jnp.float32),
        pltpu.SemaphoreType.DMA,
        pltpu.SemaphoreType.DMA,
        pltpu.SemaphoreType.DMA,
        pltpu.SemaphoreType.DMA,
        pltpu.SemaphoreType.DMA,
    ),
)(_sc_body)


def _loss_body(pos_ref, neg_ref, out_ref):
    def softplus(z):
        return jnp.maximum(z, 0.0) + jnp.log1p(jnp.exp(-jnp.abs(z)))

    total = jnp.sum(softplus(-pos_ref[...])) + jnp.sum(softplus(neg_ref[...]))
    out_ref[0, 0] = total / B


def kernel(emb, w_relation, src_pos, dst_pos, edge_type, src_neg, dst_neg):
    i32 = jnp.int32
    pos_score, neg_score = _sc_score(
        emb, w_relation, src_pos.astype(i32), dst_pos.astype(i32),
        edge_type.astype(i32), src_neg.astype(i32), dst_neg.astype(i32))

    loss = pl.pallas_call(
        _loss_body,
        out_shape=jax.ShapeDtypeStruct((1, 1), jnp.float32),
        out_specs=pl.BlockSpec(memory_space=pltpu.SMEM),
    )(pos_score.reshape(B // 128, 128), neg_score.reshape(BN // 128, 128))
    return loss[0, 0]
